# Initial kernel scaffold; baseline (speedup 1.0000x reference)
import jax
import jax.numpy as jnp
from jax.experimental import pallas as pl


def kernel(ids, offsets, table, W1, b1, W2, b2):
    raise NotImplementedError("write your pallas kernel here")


# speedup vs baseline: 1.0214x; 1.0214x over previous
import functools

import jax
import jax.numpy as jnp
from jax import lax
from jax.experimental import pallas as pl
from jax.experimental.pallas import tpu as pltpu
from jax.experimental.pallas import tpu_sc as plsc


def _mlp_body(p_ref, cnt_ref, w1_ref, b1_ref, w2_ref, b2_ref, out_ref):
    sums = p_ref[0] + p_ref[1]
    emb = sums / jnp.maximum(cnt_ref[...], 1.0)
    h = jnp.dot(emb, w1_ref[...], preferred_element_type=jnp.float32)
    h = jnp.maximum(h + b1_ref[...], 0.0)
    out = jnp.dot(h, w2_ref[...], preferred_element_type=jnp.float32)
    out_ref[...] = out + b2_ref[...]


def kernel(ids, offsets, table, W1, b1, W2, b2):
    n = ids.shape[0]
    batch = offsets.shape[0]
    emb_dim = table.shape[1]
    hidden = W1.shape[0]
    ncls = W2.shape[0]

    pos = jnp.arange(n, dtype=offsets.dtype)
    bag_ids = (jnp.searchsorted(offsets, pos, side="right") - 1).astype(jnp.int32)
    next_off = jnp.concatenate([offsets[1:], jnp.array([n], dtype=offsets.dtype)])
    counts = (next_off - offsets).astype(jnp.float32)

    info = plsc.get_sparse_core_info()
    NC, NS = info.num_cores, info.num_subcores
    NW = NC * NS
    per_w = n // NW
    CH = 128
    steps = per_w // CH
    assert per_w * NW == n and steps * CH == per_w
    rows_per_tile = batch // NS

    zeros = jnp.zeros((batch, emb_dim), jnp.float32)
    mesh = plsc.VectorSubcoreMesh(core_axis_name="c", subcore_axis_name="s")

    @functools.partial(
        pl.kernel,
        mesh=mesh,
        compiler_params=pltpu.CompilerParams(use_tc_tiling_on_sc=False),
        out_type=jax.ShapeDtypeStruct((NC, batch, emb_dim), jnp.float32),
        scratch_types=[
            pltpu.VMEM((CH,), jnp.int32),
            pltpu.VMEM((CH,), jnp.int32),
            pltpu.VMEM((CH, emb_dim), jnp.float32),
            pltpu.VMEM_SHARED((batch, emb_dim), jnp.float32),
            pltpu.SemaphoreType.DMA,
        ],
    )
    def segsum(ids_hbm, bags_hbm, tab_hbm, zeros_hbm, out_hbm,
               idx_v, bag_v, rows_v, acc, sem):
        cid = lax.axis_index("c")
        sid = lax.axis_index("s")
        base = (cid * NS + sid) * per_w
        pltpu.sync_copy(zeros_hbm.at[pl.ds(sid * rows_per_tile, rows_per_tile)],
                        acc.at[pl.ds(sid * rows_per_tile, rows_per_tile)])
        plsc.subcore_barrier()

        def step_body(i, carry):
            off = base + i * CH
            pltpu.sync_copy(ids_hbm.at[pl.ds(off, CH)], idx_v)
            pltpu.sync_copy(bags_hbm.at[pl.ds(off, CH)], bag_v)
            pltpu.async_copy(tab_hbm.at[idx_v], rows_v, sem).wait()
            pltpu.sync_copy(rows_v, acc.at[bag_v], add=True)
            return carry

        lax.fori_loop(0, steps, step_body, 0)
        plsc.subcore_barrier()
        pltpu.sync_copy(acc.at[pl.ds(sid * rows_per_tile, rows_per_tile)],
                        out_hbm.at[cid, pl.ds(sid * rows_per_tile, rows_per_tile)])

    partials = segsum(ids, bag_ids, table, zeros)

    BB = 512
    out = pl.pallas_call(
        _mlp_body,
        grid=(batch // BB,),
        in_specs=[
            pl.BlockSpec((NC, BB, emb_dim), lambda i: (0, i, 0)),
            pl.BlockSpec((BB, 1), lambda i: (i, 0)),
            pl.BlockSpec((emb_dim, hidden), lambda i: (0, 0)),
            pl.BlockSpec((1, hidden), lambda i: (0, 0)),
            pl.BlockSpec((hidden, ncls), lambda i: (0, 0)),
            pl.BlockSpec((1, ncls), lambda i: (0, 0)),
        ],
        out_specs=pl.BlockSpec((BB, ncls), lambda i: (i, 0)),
        out_shape=jax.ShapeDtypeStruct((batch, ncls), jnp.float32),
    )(partials, counts[:, None], W1.T, b1[None, :], W2.T, b2[None, :])
    return out


# speedup vs baseline: 27.9237x; 27.3382x over previous
import functools

import jax
import jax.numpy as jnp
from jax import lax
from jax.experimental import pallas as pl
from jax.experimental.pallas import tpu as pltpu
from jax.experimental.pallas import tpu_sc as plsc


def _mlp_body(p_ref, cnt_ref, w1_ref, b1_ref, w2_ref, b2_ref, out_ref):
    sums = p_ref[0] + p_ref[1]
    emb = sums / jnp.maximum(cnt_ref[...], 1.0)
    h = jnp.dot(emb, w1_ref[...], preferred_element_type=jnp.float32)
    h = jnp.maximum(h + b1_ref[...], 0.0)
    out = jnp.dot(h, w2_ref[...], preferred_element_type=jnp.float32)
    out_ref[...] = out + b2_ref[...]


def kernel(ids, offsets, table, W1, b1, W2, b2):
    n = ids.shape[0]
    batch = offsets.shape[0]
    emb_dim = table.shape[1]
    hidden = W1.shape[0]
    ncls = W2.shape[0]

    ind = jnp.zeros((n,), jnp.int32).at[offsets].add(1, mode="drop")
    bag_ids = jnp.cumsum(ind) - 1
    next_off = jnp.concatenate([offsets[1:], jnp.array([n], dtype=offsets.dtype)])
    counts = (next_off - offsets).astype(jnp.float32)

    info = plsc.get_sparse_core_info()
    NC, NS = info.num_cores, info.num_subcores
    NW = NC * NS
    per_w = n // NW
    CH = 128
    steps = per_w // CH
    assert per_w * NW == n and steps * CH == per_w
    rows_per_tile = batch // NS

    zeros = jnp.zeros((batch, emb_dim), jnp.float32)
    mesh = plsc.VectorSubcoreMesh(core_axis_name="c", subcore_axis_name="s")

    @functools.partial(
        pl.kernel,
        mesh=mesh,
        compiler_params=pltpu.CompilerParams(use_tc_tiling_on_sc=False),
        out_type=jax.ShapeDtypeStruct((NC, batch, emb_dim), jnp.float32),
        scratch_types=[
            pltpu.VMEM((CH,), jnp.int32),
            pltpu.VMEM((CH,), jnp.int32),
            pltpu.VMEM((CH, emb_dim), jnp.float32),
            pltpu.VMEM_SHARED((batch, emb_dim), jnp.float32),
            pltpu.SemaphoreType.DMA,
        ],
    )
    def segsum(ids_hbm, bags_hbm, tab_hbm, zeros_hbm, out_hbm,
               idx_v, bag_v, rows_v, acc, sem):
        cid = lax.axis_index("c")
        sid = lax.axis_index("s")
        base = (cid * NS + sid) * per_w
        pltpu.sync_copy(zeros_hbm.at[pl.ds(sid * rows_per_tile, rows_per_tile)],
                        acc.at[pl.ds(sid * rows_per_tile, rows_per_tile)])
        plsc.subcore_barrier()

        def step_body(i, carry):
            off = base + i * CH
            pltpu.sync_copy(ids_hbm.at[pl.ds(off, CH)], idx_v)
            pltpu.sync_copy(bags_hbm.at[pl.ds(off, CH)], bag_v)
            pltpu.async_copy(tab_hbm.at[idx_v], rows_v, sem).wait()
            pltpu.sync_copy(rows_v, acc.at[bag_v], add=True)
            return carry

        lax.fori_loop(0, steps, step_body, 0)
        plsc.subcore_barrier()
        pltpu.sync_copy(acc.at[pl.ds(sid * rows_per_tile, rows_per_tile)],
                        out_hbm.at[cid, pl.ds(sid * rows_per_tile, rows_per_tile)])

    partials = segsum(ids, bag_ids, table, zeros)

    BB = 512
    out = pl.pallas_call(
        _mlp_body,
        grid=(batch // BB,),
        in_specs=[
            pl.BlockSpec((NC, BB, emb_dim), lambda i: (0, i, 0)),
            pl.BlockSpec((BB, 1), lambda i: (i, 0)),
            pl.BlockSpec((emb_dim, hidden), lambda i: (0, 0)),
            pl.BlockSpec((1, hidden), lambda i: (0, 0)),
            pl.BlockSpec((hidden, ncls), lambda i: (0, 0)),
            pl.BlockSpec((1, ncls), lambda i: (0, 0)),
        ],
        out_specs=pl.BlockSpec((BB, ncls), lambda i: (i, 0)),
        out_shape=jax.ShapeDtypeStruct((batch, ncls), jnp.float32),
    )(partials, counts[:, None], W1.T, b1[None, :], W2.T, b2[None, :])
    return out


# speedup vs baseline: 27.9247x; 1.0000x over previous
import functools

import jax
import jax.numpy as jnp
from jax import lax
from jax.experimental import pallas as pl
from jax.experimental.pallas import tpu as pltpu
from jax.experimental.pallas import tpu_sc as plsc

_LANES = 128


def _shift_down_rows(p, sh):
    z = jnp.zeros((sh,) + p.shape[1:], p.dtype)
    return jnp.concatenate([z, p[:-sh]], axis=0)


def _bag_ids_body(ind_ref, out_ref):
    x = ind_ref[...]
    r = x.shape[0]
    p = x
    sh = 1
    while sh < _LANES:
        z = jnp.zeros((p.shape[0], sh), p.dtype)
        p = p + jnp.concatenate([z, p[:, :-sh]], axis=1)
        sh *= 2
    rowtot = p[:, _LANES - 1:]
    p2 = rowtot
    sh = 1
    while sh < r:
        p2 = p2 + _shift_down_rows(p2, sh)
        sh *= 2
    out_ref[...] = p + (p2 - rowtot) - 1


def _mlp_body(p_ref, cnt_ref, w1_ref, b1_ref, w2_ref, b2_ref, out_ref):
    sums = p_ref[0] + p_ref[1]
    emb = sums / jnp.maximum(cnt_ref[...], 1.0)
    h = jnp.dot(emb, w1_ref[...], preferred_element_type=jnp.float32)
    h = jnp.maximum(h + b1_ref[...], 0.0)
    out = jnp.dot(h, w2_ref[...], preferred_element_type=jnp.float32)
    out_ref[...] = out + b2_ref[...]


def kernel(ids, offsets, table, W1, b1, W2, b2):
    n = ids.shape[0]
    batch = offsets.shape[0]
    emb_dim = table.shape[1]
    hidden = W1.shape[0]
    ncls = W2.shape[0]

    ind = jnp.zeros((n,), jnp.int32).at[offsets].add(1, mode="drop")
    rows_2d = n // _LANES
    bag2d = pl.pallas_call(
        _bag_ids_body,
        out_shape=jax.ShapeDtypeStruct((rows_2d, _LANES), jnp.int32),
    )(ind.reshape(rows_2d, _LANES))

    next_off = jnp.concatenate([offsets[1:], jnp.array([n], dtype=offsets.dtype)])
    counts = (next_off - offsets).astype(jnp.float32)

    info = plsc.get_sparse_core_info()
    NC, NS = info.num_cores, info.num_subcores
    NW = NC * NS
    per_w = n // NW
    CH = 128
    steps = per_w // CH
    assert per_w * NW == n and steps * CH == per_w and steps % 2 == 0
    rows_per_tile = batch // NS

    zeros = jnp.zeros((batch, emb_dim), jnp.float32)
    ids2d = ids.reshape(n // CH, CH)
    mesh = plsc.VectorSubcoreMesh(core_axis_name="c", subcore_axis_name="s")

    @functools.partial(
        pl.kernel,
        mesh=mesh,
        compiler_params=pltpu.CompilerParams(use_tc_tiling_on_sc=False),
        out_type=jax.ShapeDtypeStruct((NC, batch, emb_dim), jnp.float32),
        scratch_types=[
            pltpu.VMEM((steps, CH), jnp.int32),
            pltpu.VMEM((steps, CH), jnp.int32),
            pltpu.VMEM((CH, emb_dim), jnp.float32),
            pltpu.VMEM((CH, emb_dim), jnp.float32),
            pltpu.VMEM_SHARED((batch, emb_dim), jnp.float32),
            pltpu.SemaphoreType.DMA,
            pltpu.SemaphoreType.DMA,
        ],
    )
    def segsum(ids_hbm, bags_hbm, tab_hbm, zeros_hbm, out_hbm,
               idx_v, bag_v, rows0, rows1, acc, sem0, sem1):
        cid = lax.axis_index("c")
        sid = lax.axis_index("s")
        wid = cid * NS + sid
        pltpu.sync_copy(ids_hbm.at[pl.ds(wid * steps, steps)], idx_v)
        pltpu.sync_copy(bags_hbm.at[pl.ds(wid * steps, steps)], bag_v)
        pltpu.sync_copy(zeros_hbm.at[pl.ds(sid * rows_per_tile, rows_per_tile)],
                        acc.at[pl.ds(sid * rows_per_tile, rows_per_tile)])
        plsc.subcore_barrier()

        pltpu.async_copy(tab_hbm.at[idx_v.at[0]], rows0, sem0)

        def pair_body(j, carry):
            k0 = 2 * j
            pltpu.async_copy(tab_hbm.at[idx_v.at[k0 + 1]], rows1, sem1)
            pltpu.make_async_copy(tab_hbm.at[idx_v.at[k0]], rows0, sem0).wait()
            pltpu.sync_copy(rows0, acc.at[bag_v.at[k0]], add=True)
            k2 = jnp.minimum(k0 + 2, steps - 1)
            pltpu.async_copy(tab_hbm.at[idx_v.at[k2]], rows0, sem0)
            pltpu.make_async_copy(tab_hbm.at[idx_v.at[k0 + 1]], rows1, sem1).wait()
            pltpu.sync_copy(rows1, acc.at[bag_v.at[k0 + 1]], add=True)
            return carry

        lax.fori_loop(0, steps // 2, pair_body, 0)
        pltpu.make_async_copy(tab_hbm.at[idx_v.at[0]], rows0, sem0).wait()
        plsc.subcore_barrier()
        pltpu.sync_copy(acc.at[pl.ds(sid * rows_per_tile, rows_per_tile)],
                        out_hbm.at[cid, pl.ds(sid * rows_per_tile, rows_per_tile)])

    partials = segsum(ids2d, bag2d, table, zeros)

    BB = 512
    out = pl.pallas_call(
        _mlp_body,
        grid=(batch // BB,),
        in_specs=[
            pl.BlockSpec((NC, BB, emb_dim), lambda i: (0, i, 0)),
            pl.BlockSpec((BB, 1), lambda i: (i, 0)),
            pl.BlockSpec((emb_dim, hidden), lambda i: (0, 0)),
            pl.BlockSpec((1, hidden), lambda i: (0, 0)),
            pl.BlockSpec((hidden, ncls), lambda i: (0, 0)),
            pl.BlockSpec((1, ncls), lambda i: (0, 0)),
        ],
        out_specs=pl.BlockSpec((BB, ncls), lambda i: (i, 0)),
        out_shape=jax.ShapeDtypeStruct((batch, ncls), jnp.float32),
    )(partials, counts[:, None], W1.T, b1[None, :], W2.T, b2[None, :])
    return out


# speedup vs baseline: 30.4027x; 1.0887x over previous
import functools

import jax
import jax.numpy as jnp
from jax import lax
from jax.experimental import pallas as pl
from jax.experimental.pallas import tpu as pltpu
from jax.experimental.pallas import tpu_sc as plsc

_LANES = 128


def _shift_down_rows(p, sh):
    z = jnp.zeros((sh,) + p.shape[1:], p.dtype)
    return jnp.concatenate([z, p[:-sh]], axis=0)


def _bag_ids_body(ind_ref, out_ref):
    x = ind_ref[...]
    r = x.shape[0]
    p = x
    sh = 1
    while sh < _LANES:
        z = jnp.zeros((p.shape[0], sh), p.dtype)
        p = p + jnp.concatenate([z, p[:, :-sh]], axis=1)
        sh *= 2
    rowtot = p[:, _LANES - 1:]
    p2 = rowtot
    sh = 1
    while sh < r:
        p2 = p2 + _shift_down_rows(p2, sh)
        sh *= 2
    out_ref[...] = p + (p2 - rowtot) - 1


def _mlp_body(p_ref, cnt_ref, w1_ref, b1_ref, w2_ref, b2_ref, out_ref):
    sums = p_ref[0, :, :64] + p_ref[1, :, :64]
    emb = sums / jnp.maximum(cnt_ref[...], 1.0)
    h = jnp.dot(emb, w1_ref[...], preferred_element_type=jnp.float32)
    h = jnp.maximum(h + b1_ref[...], 0.0)
    out = jnp.dot(h, w2_ref[...], preferred_element_type=jnp.float32)
    out_ref[...] = out + b2_ref[...]


def kernel(ids, offsets, table, W1, b1, W2, b2):
    n = ids.shape[0]
    batch = offsets.shape[0]
    emb_dim = table.shape[1]
    hidden = W1.shape[0]
    ncls = W2.shape[0]

    ind = jnp.zeros((n,), jnp.int32).at[offsets].add(1, mode="drop")
    rows_2d = n // _LANES
    bag2d = pl.pallas_call(
        _bag_ids_body,
        out_shape=jax.ShapeDtypeStruct((rows_2d, _LANES), jnp.int32),
    )(ind.reshape(rows_2d, _LANES))

    next_off = jnp.concatenate([offsets[1:], jnp.array([n], dtype=offsets.dtype)])
    counts = (next_off - offsets).astype(jnp.float32)

    info = plsc.get_sparse_core_info()
    NC, NS = info.num_cores, info.num_subcores
    NW = NC * NS
    per_w = n // NW
    CH = 128
    steps = per_w // CH
    assert per_w * NW == n and steps * CH == per_w and steps % 2 == 0
    rows_per_tile = batch // NS

    table128 = jnp.pad(table, ((0, 0), (0, _LANES - emb_dim)))
    zeros = jnp.zeros((batch, _LANES), jnp.float32)
    ids3d = ids.reshape(NW, steps, CH)
    bags3d = bag2d.reshape(NW, steps, CH)
    mesh = plsc.VectorSubcoreMesh(core_axis_name="c", subcore_axis_name="s")

    @functools.partial(
        pl.kernel,
        mesh=mesh,
        out_type=jax.ShapeDtypeStruct((NC, batch, _LANES), jnp.float32),
        scratch_types=[
            pltpu.VMEM((steps, CH), jnp.int32),
            pltpu.VMEM((steps, CH), jnp.int32),
            pltpu.VMEM((CH, _LANES), jnp.float32),
            pltpu.VMEM((CH, _LANES), jnp.float32),
            pltpu.VMEM_SHARED((batch, _LANES), jnp.float32),
            pltpu.SemaphoreType.DMA,
            pltpu.SemaphoreType.DMA,
        ],
    )
    def segsum(ids_hbm, bags_hbm, tab_hbm, zeros_hbm, out_hbm,
               idx_v, bag_v, rows0, rows1, acc, sem0, sem1):
        cid = lax.axis_index("c")
        sid = lax.axis_index("s")
        wid = cid * NS + sid
        pltpu.sync_copy(ids_hbm.at[wid], idx_v)
        pltpu.sync_copy(bags_hbm.at[wid], bag_v)
        pltpu.sync_copy(zeros_hbm.at[pl.ds(sid * rows_per_tile, rows_per_tile)],
                        acc.at[pl.ds(sid * rows_per_tile, rows_per_tile)])
        plsc.subcore_barrier()

        pltpu.async_copy(tab_hbm.at[idx_v.at[0]], rows0, sem0)

        def pair_body(j, carry):
            k0 = 2 * j
            pltpu.async_copy(tab_hbm.at[idx_v.at[k0 + 1]], rows1, sem1)
            pltpu.make_async_copy(tab_hbm.at[idx_v.at[k0]], rows0, sem0).wait()
            pltpu.sync_copy(rows0, acc.at[bag_v.at[k0]], add=True)
            k2 = jnp.minimum(k0 + 2, steps - 1)
            pltpu.async_copy(tab_hbm.at[idx_v.at[k2]], rows0, sem0)
            pltpu.make_async_copy(tab_hbm.at[idx_v.at[k0 + 1]], rows1, sem1).wait()
            pltpu.sync_copy(rows1, acc.at[bag_v.at[k0 + 1]], add=True)
            return carry

        lax.fori_loop(0, steps // 2, pair_body, 0)
        pltpu.make_async_copy(tab_hbm.at[idx_v.at[0]], rows0, sem0).wait()
        plsc.subcore_barrier()
        pltpu.sync_copy(acc.at[pl.ds(sid * rows_per_tile, rows_per_tile)],
                        out_hbm.at[cid, pl.ds(sid * rows_per_tile, rows_per_tile)])

    partials = segsum(ids3d, bags3d, table128, zeros)

    BB = 512
    out = pl.pallas_call(
        _mlp_body,
        grid=(batch // BB,),
        in_specs=[
            pl.BlockSpec((NC, BB, _LANES), lambda i: (0, i, 0)),
            pl.BlockSpec((BB, 1), lambda i: (i, 0)),
            pl.BlockSpec((emb_dim, hidden), lambda i: (0, 0)),
            pl.BlockSpec((1, hidden), lambda i: (0, 0)),
            pl.BlockSpec((hidden, ncls), lambda i: (0, 0)),
            pl.BlockSpec((1, ncls), lambda i: (0, 0)),
        ],
        out_specs=pl.BlockSpec((BB, ncls), lambda i: (i, 0)),
        out_shape=jax.ShapeDtypeStruct((batch, ncls), jnp.float32),
    )(partials, counts[:, None], W1.T, b1[None, :], W2.T, b2[None, :])
    return out


# speedup vs baseline: 50.1360x; 1.6491x over previous
import functools

import jax
import jax.numpy as jnp
from jax import lax
from jax.experimental import pallas as pl
from jax.experimental.pallas import tpu as pltpu
from jax.experimental.pallas import tpu_sc as plsc

_LANES = 128


def _shift_down_rows(p, sh):
    z = jnp.zeros((sh,) + p.shape[1:], p.dtype)
    return jnp.concatenate([z, p[:-sh]], axis=0)


def _bag_ids_body(ind_ref, out_ref):
    x = ind_ref[...]
    r = x.shape[0]
    p = x
    sh = 1
    while sh < _LANES:
        z = jnp.zeros((p.shape[0], sh), p.dtype)
        p = p + jnp.concatenate([z, p[:, :-sh]], axis=1)
        sh *= 2
    rowtot = p[:, _LANES - 1:]
    p2 = rowtot
    sh = 1
    while sh < r:
        p2 = p2 + _shift_down_rows(p2, sh)
        sh *= 2
    out_ref[...] = p + (p2 - rowtot) - 1


def _transpose_pad_body(t_ref, out_ref):
    x = t_ref[...]
    xt = x.T
    z = jnp.zeros_like(xt)
    out_ref[...] = jnp.concatenate([xt, z], axis=1)


def _mlp_body(p_ref, cnt_ref, w1_ref, b1_ref, w2_ref, b2_ref, out_ref):
    sums = p_ref[0, :, :64] + p_ref[1, :, :64]
    emb = sums / jnp.maximum(cnt_ref[...], 1.0)
    h = jnp.dot(emb, w1_ref[...], preferred_element_type=jnp.float32)
    h = jnp.maximum(h + b1_ref[...], 0.0)
    out = jnp.dot(h, w2_ref[...], preferred_element_type=jnp.float32)
    out_ref[...] = out + b2_ref[...]


def kernel(ids, offsets, table, W1, b1, W2, b2):
    n = ids.shape[0]
    batch = offsets.shape[0]
    emb_dim = table.shape[1]
    hidden = W1.shape[0]
    ncls = W2.shape[0]

    ind = jnp.zeros((n,), jnp.int32).at[offsets].add(1, mode="drop")
    rows_2d = n // _LANES
    bag2d = pl.pallas_call(
        _bag_ids_body,
        out_shape=jax.ShapeDtypeStruct((rows_2d, _LANES), jnp.int32),
    )(ind.reshape(rows_2d, _LANES))

    next_off = jnp.concatenate([offsets[1:], jnp.array([n], dtype=offsets.dtype)])
    counts = (next_off - offsets).astype(jnp.float32)

    info = plsc.get_sparse_core_info()
    NC, NS = info.num_cores, info.num_subcores
    NW = NC * NS
    per_w = n // NW
    CH = 128
    steps = per_w // CH
    assert per_w * NW == n and steps * CH == per_w and steps % 2 == 0
    rows_per_tile = batch // NS

    TB = 16384
    vocab = table.shape[0]
    nblk = (vocab + TB - 1) // TB
    table128 = pl.pallas_call(
        _transpose_pad_body,
        grid=(nblk,),
        in_specs=[pl.BlockSpec((emb_dim, TB), lambda i: (0, i))],
        out_specs=pl.BlockSpec((TB, _LANES), lambda i: (i, 0)),
        out_shape=jax.ShapeDtypeStruct((nblk * TB, _LANES), jnp.float32),
    )(table.T)
    zeros = jnp.zeros((batch, _LANES), jnp.float32)
    ids3d = ids.reshape(NW, steps, CH)
    bags3d = bag2d.reshape(NW, steps, CH)
    mesh = plsc.VectorSubcoreMesh(core_axis_name="c", subcore_axis_name="s")

    @functools.partial(
        pl.kernel,
        mesh=mesh,
        out_type=jax.ShapeDtypeStruct((NC, batch, _LANES), jnp.float32),
        scratch_types=[
            pltpu.VMEM((steps, CH), jnp.int32),
            pltpu.VMEM((steps, CH), jnp.int32),
            pltpu.VMEM((CH, _LANES), jnp.float32),
            pltpu.VMEM((CH, _LANES), jnp.float32),
            pltpu.VMEM_SHARED((batch, _LANES), jnp.float32),
            pltpu.SemaphoreType.DMA,
            pltpu.SemaphoreType.DMA,
        ],
    )
    def segsum(ids_hbm, bags_hbm, tab_hbm, zeros_hbm, out_hbm,
               idx_v, bag_v, rows0, rows1, acc, sem0, sem1):
        cid = lax.axis_index("c")
        sid = lax.axis_index("s")
        wid = cid * NS + sid
        pltpu.sync_copy(ids_hbm.at[wid], idx_v)
        pltpu.sync_copy(bags_hbm.at[wid], bag_v)
        pltpu.sync_copy(zeros_hbm.at[pl.ds(sid * rows_per_tile, rows_per_tile)],
                        acc.at[pl.ds(sid * rows_per_tile, rows_per_tile)])
        plsc.subcore_barrier()

        pltpu.async_copy(tab_hbm.at[idx_v.at[0]], rows0, sem0)

        def pair_body(j, carry):
            k0 = 2 * j
            pltpu.async_copy(tab_hbm.at[idx_v.at[k0 + 1]], rows1, sem1)
            pltpu.make_async_copy(tab_hbm.at[idx_v.at[k0]], rows0, sem0).wait()
            pltpu.sync_copy(rows0, acc.at[bag_v.at[k0]], add=True)
            k2 = jnp.minimum(k0 + 2, steps - 1)
            pltpu.async_copy(tab_hbm.at[idx_v.at[k2]], rows0, sem0)
            pltpu.make_async_copy(tab_hbm.at[idx_v.at[k0 + 1]], rows1, sem1).wait()
            pltpu.sync_copy(rows1, acc.at[bag_v.at[k0 + 1]], add=True)
            return carry

        lax.fori_loop(0, steps // 2, pair_body, 0)
        pltpu.make_async_copy(tab_hbm.at[idx_v.at[0]], rows0, sem0).wait()
        plsc.subcore_barrier()
        pltpu.sync_copy(acc.at[pl.ds(sid * rows_per_tile, rows_per_tile)],
                        out_hbm.at[cid, pl.ds(sid * rows_per_tile, rows_per_tile)])

    partials = segsum(ids3d, bags3d, table128, zeros)

    BB = 512
    out = pl.pallas_call(
        _mlp_body,
        grid=(batch // BB,),
        in_specs=[
            pl.BlockSpec((NC, BB, _LANES), lambda i: (0, i, 0)),
            pl.BlockSpec((BB, 1), lambda i: (i, 0)),
            pl.BlockSpec((emb_dim, hidden), lambda i: (0, 0)),
            pl.BlockSpec((1, hidden), lambda i: (0, 0)),
            pl.BlockSpec((hidden, ncls), lambda i: (0, 0)),
            pl.BlockSpec((1, ncls), lambda i: (0, 0)),
        ],
        out_specs=pl.BlockSpec((BB, ncls), lambda i: (i, 0)),
        out_shape=jax.ShapeDtypeStruct((batch, ncls), jnp.float32),
    )(partials, counts[:, None], W1.T, b1[None, :], W2.T, b2[None, :])
    return out


# speedup vs baseline: 58.2746x; 1.1623x over previous
import functools

import jax
import jax.numpy as jnp
from jax import lax
from jax.experimental import pallas as pl
from jax.experimental.pallas import tpu as pltpu
from jax.experimental.pallas import tpu_sc as plsc

_LANES = 128


def _shift_down_rows(p, sh):
    z = jnp.zeros((sh,) + p.shape[1:], p.dtype)
    return jnp.concatenate([z, p[:-sh]], axis=0)


def _bag_ids_body(ind_ref, out_ref):
    x = ind_ref[...]
    r = x.shape[0]
    p = x
    sh = 1
    while sh < _LANES:
        z = jnp.zeros((p.shape[0], sh), p.dtype)
        p = p + jnp.concatenate([z, p[:, :-sh]], axis=1)
        sh *= 2
    rowtot = p[:, _LANES - 1:]
    p2 = rowtot
    sh = 1
    while sh < r:
        p2 = p2 + _shift_down_rows(p2, sh)
        sh *= 2
    out_ref[...] = p + (p2 - rowtot) - 1


def _transpose_pad_body(t_ref, out_ref):
    x = t_ref[...]
    xt = x.T
    z = jnp.zeros_like(xt)
    out_ref[...] = jnp.concatenate([xt, z], axis=1)


def _mlp_body(p_ref, cnt_ref, w1_ref, b1_ref, w2_ref, b2_ref, out_ref):
    sums = p_ref[0, :, :64] + p_ref[1, :, :64]
    emb = sums / jnp.maximum(cnt_ref[...], 1.0)
    h = jnp.dot(emb, w1_ref[...], preferred_element_type=jnp.float32)
    h = jnp.maximum(h + b1_ref[...], 0.0)
    out = jnp.dot(h, w2_ref[...], preferred_element_type=jnp.float32)
    out_ref[...] = out + b2_ref[...]


def kernel(ids, offsets, table, W1, b1, W2, b2):
    n = ids.shape[0]
    batch = offsets.shape[0]
    emb_dim = table.shape[1]
    hidden = W1.shape[0]
    ncls = W2.shape[0]

    ind = jnp.zeros((n,), jnp.int32).at[offsets].add(1, mode="drop")
    rows_2d = n // _LANES
    bag2d = pl.pallas_call(
        _bag_ids_body,
        out_shape=jax.ShapeDtypeStruct((rows_2d, _LANES), jnp.int32),
    )(ind.reshape(rows_2d, _LANES))

    next_off = jnp.concatenate([offsets[1:], jnp.array([n], dtype=offsets.dtype)])
    counts = (next_off - offsets).astype(jnp.float32)

    info = plsc.get_sparse_core_info()
    NC, NS = info.num_cores, info.num_subcores
    NW = NC * NS
    per_w = n // NW
    CH = 128
    steps = per_w // CH
    assert per_w * NW == n and steps * CH == per_w and steps % 2 == 0
    rows_per_tile = batch // NS

    TB = 16384
    vocab = table.shape[0]
    nblk = (vocab + TB - 1) // TB
    table128 = pl.pallas_call(
        _transpose_pad_body,
        grid=(nblk,),
        in_specs=[pl.BlockSpec((emb_dim, TB), lambda i: (0, i))],
        out_specs=pl.BlockSpec((TB, _LANES), lambda i: (i, 0)),
        out_shape=jax.ShapeDtypeStruct((nblk * TB, _LANES), jnp.float32),
    )(table.T)
    zeros = jnp.zeros((batch, _LANES), jnp.float32)
    ids3d = ids.reshape(NW, steps, CH)
    bags3d = bag2d.reshape(NW, steps, CH)
    mesh = plsc.VectorSubcoreMesh(core_axis_name="c", subcore_axis_name="s")

    @functools.partial(
        pl.kernel,
        mesh=mesh,
        out_type=jax.ShapeDtypeStruct((NC, batch, _LANES), jnp.float32),
        scratch_types=[
            pltpu.VMEM((steps, CH), jnp.int32),
            pltpu.VMEM((steps, CH), jnp.int32),
            pltpu.VMEM((CH, _LANES), jnp.float32),
            pltpu.VMEM((CH, _LANES), jnp.float32),
            pltpu.VMEM((16, _LANES), jnp.float32),
            pltpu.VMEM((16,), jnp.int32),
            pltpu.VMEM_SHARED((batch, _LANES), jnp.float32),
            pltpu.SemaphoreType.DMA,
            pltpu.SemaphoreType.DMA,
        ],
    )
    def segsum(ids_hbm, bags_hbm, tab_hbm, zeros_hbm, out_hbm,
               idx_v, bag_v, rows0, rows1, sumbuf, idx16, acc, sem0, sem1):
        cid = lax.axis_index("c")
        sid = lax.axis_index("s")
        wid = cid * NS + sid
        pltpu.sync_copy(ids_hbm.at[wid], idx_v)
        pltpu.sync_copy(bags_hbm.at[wid], bag_v)
        pltpu.sync_copy(zeros_hbm.at[pl.ds(sid * rows_per_tile, rows_per_tile)],
                        acc.at[pl.ds(sid * rows_per_tile, rows_per_tile)])
        pltpu.sync_copy(zeros_hbm.at[pl.ds(0, 16)], sumbuf)
        plsc.subcore_barrier()

        nlane = _LANES // 16

        def scatter_chunk(rbuf, k):
            bag_row = bag_v.at[k]
            v0 = bag_row[pl.ds(0, 16)]
            vL = bag_row[pl.ds(CH - 16, 16)]
            single = v0[0] == vL[15]

            @pl.when(single)
            def _():
                def rbody(r, accs):
                    row = rbuf.at[r]
                    return tuple(a + row[pl.ds(16 * c, 16)]
                                 for c, a in enumerate(accs))
                accs = lax.fori_loop(
                    0, CH, rbody,
                    tuple(jnp.zeros((16,), jnp.float32) for _ in range(nlane)))
                for c in range(nlane):
                    sumbuf[0, pl.ds(16 * c, 16)] = accs[c]
                idx16[...] = bag_row[pl.ds(0, 16)]
                pltpu.sync_copy(sumbuf, acc.at[idx16], add=True)

            @pl.when(jnp.logical_not(single))
            def _():
                pltpu.sync_copy(rbuf, acc.at[bag_v.at[k]], add=True)

        pltpu.async_copy(tab_hbm.at[idx_v.at[0]], rows0, sem0)

        def pair_body(j, carry):
            k0 = 2 * j
            pltpu.async_copy(tab_hbm.at[idx_v.at[k0 + 1]], rows1, sem1)
            pltpu.make_async_copy(tab_hbm.at[idx_v.at[k0]], rows0, sem0).wait()
            scatter_chunk(rows0, k0)
            k2 = jnp.minimum(k0 + 2, steps - 1)
            pltpu.async_copy(tab_hbm.at[idx_v.at[k2]], rows0, sem0)
            pltpu.make_async_copy(tab_hbm.at[idx_v.at[k0 + 1]], rows1, sem1).wait()
            scatter_chunk(rows1, k0 + 1)
            return carry

        lax.fori_loop(0, steps // 2, pair_body, 0)
        pltpu.make_async_copy(tab_hbm.at[idx_v.at[0]], rows0, sem0).wait()
        plsc.subcore_barrier()
        pltpu.sync_copy(acc.at[pl.ds(sid * rows_per_tile, rows_per_tile)],
                        out_hbm.at[cid, pl.ds(sid * rows_per_tile, rows_per_tile)])

    partials = segsum(ids3d, bags3d, table128, zeros)

    BB = 512
    out = pl.pallas_call(
        _mlp_body,
        grid=(batch // BB,),
        in_specs=[
            pl.BlockSpec((NC, BB, _LANES), lambda i: (0, i, 0)),
            pl.BlockSpec((BB, 1), lambda i: (i, 0)),
            pl.BlockSpec((emb_dim, hidden), lambda i: (0, 0)),
            pl.BlockSpec((1, hidden), lambda i: (0, 0)),
            pl.BlockSpec((hidden, ncls), lambda i: (0, 0)),
            pl.BlockSpec((1, ncls), lambda i: (0, 0)),
        ],
        out_specs=pl.BlockSpec((BB, ncls), lambda i: (i, 0)),
        out_shape=jax.ShapeDtypeStruct((batch, ncls), jnp.float32),
    )(partials, counts[:, None], W1.T, b1[None, :], W2.T, b2[None, :])
    return out
